# baseline (speedup 1.0000x reference)
import functools

import jax
import jax.numpy as jnp
from jax.experimental import pallas as pl
from jax.experimental.pallas import tpu as pltpu

_ANCHOR_T = 4.0
_CHUNK = 8192


def _tc_body(num_layers, num_anchors, t_ref, anchors_ref, strides_ref,
             out_ref, loss_ref):
    @pl.when(pl.program_id(0) == 0)
    def _():
        for c in range(3):
            loss_ref[c] = 0.0

    t = t_ref[...]
    for i in range(num_layers):
        st = strides_ref[i]
        s = t[2:6, :] / st
        head = jnp.concatenate([t[0:2, :], s], axis=0)
        for a in range(num_anchors):
            law = anchors_ref[i, a, 0] / st
            lah = anchors_ref[i, a, 1] / st
            rw = s[2:3, :] / law
            rh = s[3:4, :] / lah
            worst = jnp.maximum(jnp.maximum(rw, 1.0 / rw),
                                jnp.maximum(rh, 1.0 / rh))
            keep = worst < _ANCHOR_T
            block = jnp.concatenate(
                [head, jnp.full((1, t.shape[1]), float(a), jnp.float32)],
                axis=0)
            out_ref[i, a] = jnp.where(keep, block, 0.0)


@functools.lru_cache(maxsize=None)
def _build_tc_kernel(num_layers, num_anchors, num_targets):
    ncols = 7
    grid = num_targets // _CHUNK
    return pl.pallas_call(
        functools.partial(_tc_body, num_layers, num_anchors),
        grid=(grid,),
        in_specs=[
            pl.BlockSpec((6, _CHUNK), lambda k: (0, k)),
            pl.BlockSpec(memory_space=pltpu.SMEM),
            pl.BlockSpec(memory_space=pltpu.SMEM),
        ],
        out_specs=[
            pl.BlockSpec(
                (num_layers, num_anchors, ncols, _CHUNK),
                lambda k: (0, 0, 0, k)),
            pl.BlockSpec(memory_space=pltpu.SMEM),
        ],
        out_shape=[
            jax.ShapeDtypeStruct(
                (num_layers, num_anchors, ncols, num_targets), jnp.float32),
            jax.ShapeDtypeStruct((3,), jnp.float32),
        ],
        compiler_params=pltpu.CompilerParams(
            dimension_semantics=("arbitrary",)),
    )


def kernel(preds, targets, anchors, strides):
    del preds
    num_targets = targets.shape[0]
    num_layers, num_anchors = anchors.shape[0], anchors.shape[1]
    t_t = targets[:, :6].T
    tc = _build_tc_kernel(num_layers, num_anchors, num_targets)
    out_t, losses = tc(t_t, anchors, strides)
    matched = jnp.transpose(out_t, (0, 1, 3, 2))
    return (matched, losses)


# speedup vs baseline: 1.0870x; 1.0870x over previous
import functools

import jax
import jax.numpy as jnp
from jax.experimental import pallas as pl
from jax.experimental.pallas import tpu as pltpu

_ANCHOR_T = 4.0
_CHUNK = 4096


def _tc_body(num_layers, num_anchors, t_ref, anchors_ref, strides_ref,
             out_ref, loss_ref):
    @pl.when(pl.program_id(0) == 0)
    def _():
        for c in range(3):
            loss_ref[c] = 0.0

    t = t_ref[...]
    for i in range(num_layers):
        st = strides_ref[i]
        s = t[2:6, :] / st
        head = jnp.concatenate([t[0:2, :], s], axis=0)
        for a in range(num_anchors):
            law = anchors_ref[i, a, 0] / st
            lah = anchors_ref[i, a, 1] / st
            rw = s[2:3, :] / law
            rh = s[3:4, :] / lah
            worst = jnp.maximum(jnp.maximum(rw, 1.0 / rw),
                                jnp.maximum(rh, 1.0 / rh))
            keep = worst < _ANCHOR_T
            block = jnp.concatenate(
                [head, jnp.full((1, t.shape[1]), float(a), jnp.float32)],
                axis=0)
            out_ref[i, a] = jnp.where(keep, block, 0.0)


@functools.lru_cache(maxsize=None)
def _build_tc_kernel(num_layers, num_anchors, num_targets):
    ncols = 7
    grid = num_targets // _CHUNK
    return pl.pallas_call(
        functools.partial(_tc_body, num_layers, num_anchors),
        grid=(grid,),
        in_specs=[
            pl.BlockSpec((6, _CHUNK), lambda k: (0, k)),
            pl.BlockSpec(memory_space=pltpu.SMEM),
            pl.BlockSpec(memory_space=pltpu.SMEM),
        ],
        out_specs=[
            pl.BlockSpec(
                (num_layers, num_anchors, ncols, _CHUNK),
                lambda k: (0, 0, 0, k)),
            pl.BlockSpec(memory_space=pltpu.SMEM),
        ],
        out_shape=[
            jax.ShapeDtypeStruct(
                (num_layers, num_anchors, ncols, num_targets), jnp.float32),
            jax.ShapeDtypeStruct((3,), jnp.float32),
        ],
        compiler_params=pltpu.CompilerParams(
            dimension_semantics=("arbitrary",)),
    )


def kernel(preds, targets, anchors, strides):
    del preds
    num_targets = targets.shape[0]
    num_layers, num_anchors = anchors.shape[0], anchors.shape[1]
    t_t = targets[:, :6].T
    tc = _build_tc_kernel(num_layers, num_anchors, num_targets)
    out_t, losses = tc(t_t, anchors, strides)
    matched = jnp.transpose(out_t, (0, 1, 3, 2))
    return (matched, losses)
